# baseline (speedup 1.0000x reference)
import functools

import jax
import jax.numpy as jnp
import numpy as np
from jax import lax
from jax.experimental import pallas as pl
from jax.experimental.pallas import tpu as pltpu
from jax.experimental.pallas import tpu_sc as plsc

B = 64
N = 32768
K = 128
L = 16
NSLICES = N // L
UNROLL = 16
HBUCKETS = 2048
MIN32 = np.int32(-(2 ** 31))

_MESH = plsc.VectorSubcoreMesh(
    core_axis_name="c", subcore_axis_name="s", num_cores=2, num_subcores=16
)
NW = 2 * 16
ROWS_PER_W = B // NW


def _keys(xv):
    u = lax.bitcast_convert_type(xv, jnp.int32)
    kb = u ^ ((u >> 31) | MIN32)
    ki = kb ^ MIN32
    return ki, kb


def _zero_hist(hist, nslices):
    zeros = jnp.zeros((L,), jnp.int32)

    def it(i, carry):
        hist[pl.ds(i * L, L)] = zeros
        return carry

    lax.fori_loop(0, nslices, it, np.int32(0))


def _hist_pass_a(xbuf, hist):
    ones = jnp.ones((L,), jnp.int32)

    def it(i, carry):
        for u in range(UNROLL):
            s = i * UNROLL + u
            xv = xbuf[pl.ds(s * L, L)]
            _, kb = _keys(xv)
            bucket = lax.shift_right_logical(kb, 21)
            plsc.addupdate_scatter(hist, [bucket], ones)
        return carry

    lax.fori_loop(0, NSLICES // UNROLL, it, np.int32(0))


def _scan_hist(hist, need, nslices):
    iota = lax.iota(jnp.int32, L)

    def it(i, carry):
        found, b, c_above, acc = carry
        j = np.int32(nslices - 1) - i
        h = hist[pl.ds(j * L, L)]
        s = jnp.sum(h)
        incl = plsc.cumsum(h)
        suffix = acc + (s - incl) + h
        hit = jnp.logical_and(found == 0, (acc + s) >= need)
        mv = suffix >= need
        b_in = jnp.sum(jnp.where(mv, 1, 0)) - 1
        strict = suffix - h
        c_new = jnp.sum(jnp.where(iota == b_in, strict, 0))
        b = jnp.where(hit, j * L + b_in, b)
        c_above = jnp.where(hit, c_new, c_above)
        found = jnp.where(hit, np.int32(1), found)
        return found, b, c_above, acc + s

    z = np.int32(0)
    _, b, c_above, _ = lax.fori_loop(0, nslices, it, (z, z, z, z))
    return b, c_above


def _write_compact_pass(xbuf, obuf, cand, b1):
    iota = lax.iota(jnp.int32, L)

    def it(i, off_v):
        for u in range(UNROLL):
            s = i * UNROLL + u
            xv = xbuf[pl.ds(s * L, L)]
            _, kb = _keys(xv)
            bucket = lax.shift_right_logical(kb, 21)
            win = bucket > b1
            res = jnp.where(win, jnp.maximum(xv, np.float32(0.0)), np.float32(0.0))
            obuf[pl.ds(s * L, L)] = res
            pm = bucket == b1
            incl = plsc.cumsum(jnp.where(pm, np.int32(1), np.int32(0)))
            idx = off_v + incl - 1
            plsc.store_scatter(cand, [idx], s * L + iota, mask=pm)
            off_v = off_v + plsc.all_reduce_population_count(pm)
        return off_v

    off_v = lax.fori_loop(0, NSLICES // UNROLL, it, jnp.zeros((L,), jnp.int32))
    return off_v


def _cand_hist(xbuf, cand, hist, trip, nc_v, shift, bmask, prefix_shift, prefix_val):
    iota = lax.iota(jnp.int32, L)
    ones = jnp.ones((L,), jnp.int32)

    def it(t, carry):
        valid = (t * L + iota) < nc_v
        posv = cand[pl.ds(t * L, L)]
        xg = plsc.load_gather(xbuf, [jnp.where(valid, posv, np.int32(0))])
        _, kb = _keys(xg)
        bucket = lax.shift_right_logical(kb, shift) & np.int32(bmask)
        if prefix_shift is None:
            m = valid
        else:
            pv = lax.shift_right_logical(kb, prefix_shift) & np.int32(0x7FF)
            m = jnp.logical_and(valid, pv == prefix_val)
        plsc.addupdate_scatter(hist, [bucket], ones, mask=m)
        return carry

    lax.fori_loop(0, trip, it, np.int32(0))


def _fixup_pass(xbuf, obuf, cand, trip, nc_v, tsig, quota):
    iota = lax.iota(jnp.int32, L)
    qv = jnp.full((L,), quota, jnp.int32)

    def it(t, rv):
        valid = (t * L + iota) < nc_v
        posv = cand[pl.ds(t * L, L)]
        xg = plsc.load_gather(xbuf, [jnp.where(valid, posv, np.int32(0))])
        ki, _ = _keys(xg)
        gt = jnp.logical_and(valid, ki > tsig)
        eq = jnp.logical_and(valid, ki == tsig)
        incl = plsc.cumsum(jnp.where(eq, np.int32(1), np.int32(0)))
        take = jnp.logical_or(gt, jnp.logical_and(eq, (rv + incl) <= qv))
        val = jnp.maximum(xg, np.float32(0.0))
        plsc.store_scatter(obuf, [posv], val, mask=take)
        return rv + plsc.all_reduce_population_count(eq)

    lax.fori_loop(0, trip, it, jnp.zeros((L,), jnp.int32))


@functools.partial(
    pl.kernel,
    out_type=jax.ShapeDtypeStruct((B, N), jnp.float32),
    mesh=_MESH,
    compiler_params=pltpu.CompilerParams(needs_layout_passes=False),
    scratch_types=[
        pltpu.VMEM((N,), jnp.float32),
        pltpu.VMEM((N,), jnp.float32),
        pltpu.VMEM((N,), jnp.int32),
        pltpu.VMEM((HBUCKETS,), jnp.int32),
    ],
)
def _topk_sc(x_hbm, o_hbm, xbuf, obuf, cand, hist):
    wid = lax.axis_index("s") * 2 + lax.axis_index("c")
    for r in range(ROWS_PER_W):
        row = wid * ROWS_PER_W + r
        pltpu.sync_copy(x_hbm.at[row], xbuf)

        _zero_hist(hist, HBUCKETS // L)
        _hist_pass_a(xbuf, hist)
        b1, c1 = _scan_hist(hist, np.int32(K), HBUCKETS // L)

        nc_v = _write_compact_pass(xbuf, obuf, cand, b1)
        nc = jnp.max(nc_v)
        trip = (nc + np.int32(L - 1)) >> 4

        _zero_hist(hist, HBUCKETS // L)
        _cand_hist(xbuf, cand, hist, trip, nc_v, 10, 0x7FF, None, None)
        b2, c2 = _scan_hist(hist, np.int32(K) - c1, HBUCKETS // L)

        _zero_hist(hist, 1024 // L)
        _cand_hist(xbuf, cand, hist, trip, nc_v, 0, 0x3FF, 10, b2)
        b3, c3 = _scan_hist(hist, np.int32(K) - c1 - c2, 1024 // L)

        kb_t = (b1 << 21) | (b2 << 10) | b3
        tsig = kb_t ^ MIN32
        quota = np.int32(K) - (c1 + c2 + c3)

        _fixup_pass(xbuf, obuf, cand, trip, nc_v, tsig, quota)
        pltpu.sync_copy(obuf, o_hbm.at[row])


def kernel(x):
    return _topk_sc(x)


# speedup vs baseline: 1.0403x; 1.0403x over previous
import functools

import jax
import jax.numpy as jnp
import numpy as np
from jax import lax
from jax.experimental import pallas as pl
from jax.experimental.pallas import tpu as pltpu
from jax.experimental.pallas import tpu_sc as plsc

B = 64
N = 32768
K = 128
L = 16
NSLICES = N // L
UNROLL = 8
HBUCKETS = 2048
MIN32 = np.int32(-(2 ** 31))

_MESH = plsc.VectorSubcoreMesh(
    core_axis_name="c", subcore_axis_name="s", num_cores=2, num_subcores=16
)
NW = 2 * 16
ROWS_PER_W = B // NW


def _keys(xv):
    u = lax.bitcast_convert_type(xv, jnp.int32)
    kb = u ^ ((u >> 31) | MIN32)
    ki = kb ^ MIN32
    return ki, kb


def _zero_hist(hist, nslices):
    zeros = jnp.zeros((L,), jnp.int32)

    def it(i, carry):
        hist[pl.ds(i * L, L)] = zeros
        return carry

    lax.fori_loop(0, nslices, it, np.int32(0))


def _hist_pass_a(xbuf, hist):
    ones = jnp.ones((L,), jnp.int32)

    def it(i, carry):
        for u in range(UNROLL):
            s = i * UNROLL + u
            xv = xbuf[pl.ds(s * L, L)]
            _, kb = _keys(xv)
            bucket = lax.shift_right_logical(kb, 21)
            plsc.addupdate_scatter(hist, [bucket], ones)
        return carry

    lax.fori_loop(0, NSLICES // UNROLL, it, np.int32(0))


def _scan_hist(hist, need, nslices):
    iota = lax.iota(jnp.int32, L)

    def it(i, carry):
        found, b, c_above, acc = carry
        j = np.int32(nslices - 1) - i
        h = hist[pl.ds(j * L, L)]
        s = jnp.sum(h)
        incl = plsc.cumsum(h)
        suffix = acc + (s - incl) + h
        hit = jnp.logical_and(found == 0, (acc + s) >= need)
        mv = suffix >= need
        b_in = jnp.sum(jnp.where(mv, 1, 0)) - 1
        strict = suffix - h
        c_new = jnp.sum(jnp.where(iota == b_in, strict, 0))
        b = jnp.where(hit, j * L + b_in, b)
        c_above = jnp.where(hit, c_new, c_above)
        found = jnp.where(hit, np.int32(1), found)
        return found, b, c_above, acc + s

    z = np.int32(0)
    _, b, c_above, _ = lax.fori_loop(0, nslices, it, (z, z, z, z))
    return b, c_above


def _write_compact_pass(xbuf, obuf, cand, b1):
    iota = lax.iota(jnp.int32, L)

    def it(i, off_v):
        for u in range(UNROLL):
            s = i * UNROLL + u
            xv = xbuf[pl.ds(s * L, L)]
            _, kb = _keys(xv)
            bucket = lax.shift_right_logical(kb, 21)
            win = bucket > b1
            res = jnp.where(win, jnp.maximum(xv, np.float32(0.0)), np.float32(0.0))
            obuf[pl.ds(s * L, L)] = res
            pm = bucket == b1
            incl = plsc.cumsum(jnp.where(pm, np.int32(1), np.int32(0)))
            idx = off_v + incl - 1
            plsc.store_scatter(cand, [idx], s * L + iota, mask=pm)
            off_v = off_v + plsc.all_reduce_population_count(pm)
        return off_v

    off_v = lax.fori_loop(0, NSLICES // UNROLL, it, jnp.zeros((L,), jnp.int32))
    return off_v


def _cand_count(xbuf, cand, trip, nc_v, thr_v, strict):
    iota = lax.iota(jnp.int32, L)

    def it(t, cnt_v):
        valid = (t * L + iota) < nc_v
        posv = cand[pl.ds(t * L, L)]
        xg = plsc.load_gather(xbuf, [jnp.where(valid, posv, np.int32(0))])
        ki, _ = _keys(xg)
        cmp = (ki > thr_v) if strict else (ki >= thr_v)
        m = jnp.logical_and(valid, cmp)
        return cnt_v + plsc.all_reduce_population_count(m)

    return lax.fori_loop(0, trip, it, jnp.zeros((L,), jnp.int32))


def _cand_binsearch(xbuf, cand, trip, nc_v, b1, need):
    lo0 = (b1 << 21) ^ MIN32
    lo_v = jnp.full((L,), np.int32(0), jnp.int32) + lo0
    span_v = jnp.full((L,), np.int32((1 << 21) - 1), jnp.int32)
    hi_v = lo_v + span_v
    need_v = jnp.full((L,), need, jnp.int32)

    def step(i, carry):
        lo_v, hi_v = carry
        mid_v = lo_v + ((hi_v - lo_v + 1) >> 1)
        cnt_v = _cand_count(xbuf, cand, trip, nc_v, mid_v, False)
        ok = cnt_v >= need_v
        return (jnp.where(ok, mid_v, lo_v), jnp.where(ok, hi_v, mid_v - 1))

    lo_v, _ = lax.fori_loop(0, 21, step, (lo_v, hi_v))
    return lo_v


def _fixup_pass(xbuf, obuf, cand, trip, nc_v, tsig_v, qv):
    iota = lax.iota(jnp.int32, L)

    def it(t, rv):
        valid = (t * L + iota) < nc_v
        posv = cand[pl.ds(t * L, L)]
        xg = plsc.load_gather(xbuf, [jnp.where(valid, posv, np.int32(0))])
        ki, _ = _keys(xg)
        gt = jnp.logical_and(valid, ki > tsig_v)
        eq = jnp.logical_and(valid, ki == tsig_v)
        incl = plsc.cumsum(jnp.where(eq, np.int32(1), np.int32(0)))
        take = jnp.logical_or(gt, jnp.logical_and(eq, (rv + incl) <= qv))
        val = jnp.maximum(xg, np.float32(0.0))
        plsc.store_scatter(obuf, [posv], val, mask=take)
        return rv + plsc.all_reduce_population_count(eq)

    lax.fori_loop(0, trip, it, jnp.zeros((L,), jnp.int32))


@functools.partial(
    pl.kernel,
    out_type=jax.ShapeDtypeStruct((B, N), jnp.float32),
    mesh=_MESH,
    compiler_params=pltpu.CompilerParams(needs_layout_passes=False),
    scratch_types=[
        pltpu.VMEM((N,), jnp.float32),
        pltpu.VMEM((N,), jnp.float32),
        pltpu.VMEM((N,), jnp.int32),
        pltpu.VMEM((HBUCKETS,), jnp.int32),
    ],
)
def _topk_sc(x_hbm, o_hbm, xbuf, obuf, cand, hist):
    wid = lax.axis_index("s") * 2 + lax.axis_index("c")
    for r in range(ROWS_PER_W):
        row = wid * ROWS_PER_W + r
        pltpu.sync_copy(x_hbm.at[row], xbuf)

        _zero_hist(hist, HBUCKETS // L)
        _hist_pass_a(xbuf, hist)
        b1, c1 = _scan_hist(hist, np.int32(K), HBUCKETS // L)

        nc_v = _write_compact_pass(xbuf, obuf, cand, b1)
        nc = jnp.max(nc_v)
        trip = (nc + np.int32(L - 1)) >> 4

        need = np.int32(K) - c1
        tsig_v = _cand_binsearch(xbuf, cand, trip, nc_v, b1, need)
        cgt_v = _cand_count(xbuf, cand, trip, nc_v, tsig_v, True)
        qv = (jnp.zeros((L,), jnp.int32) + need) - cgt_v

        _fixup_pass(xbuf, obuf, cand, trip, nc_v, tsig_v, qv)
        pltpu.sync_copy(obuf, o_hbm.at[row])


def kernel(x):
    return _topk_sc(x)


# speedup vs baseline: 1.0897x; 1.0474x over previous
import functools

import jax
import jax.numpy as jnp
import numpy as np
from jax import lax
from jax.experimental import pallas as pl
from jax.experimental.pallas import tpu as pltpu
from jax.experimental.pallas import tpu_sc as plsc

B = 64
N = 32768
K = 128
L = 16
NSLICES = N // L
UNROLL = 8
HBUCKETS = 2048
MIN32 = np.int32(-(2 ** 31))

_MESH = plsc.VectorSubcoreMesh(
    core_axis_name="c", subcore_axis_name="s", num_cores=2, num_subcores=16
)
NW = 2 * 16
ROWS_PER_W = B // NW


def _keys(xv):
    u = lax.bitcast_convert_type(xv, jnp.int32)
    kb = u ^ ((u >> 31) | MIN32)
    ki = kb ^ MIN32
    return ki, kb


def _zero_hist(hist, nslices):
    zeros = jnp.zeros((L,), jnp.int32)

    def it(i, carry):
        hist[pl.ds(i * L, L)] = zeros
        return carry

    lax.fori_loop(0, nslices, it, np.int32(0))


def _hist_pass_a(xbuf, obuf, hist):
    ones = jnp.ones((L,), jnp.int32)
    zf = jnp.zeros((L,), jnp.float32)

    def it(i, carry):
        for u in range(UNROLL):
            s = i * UNROLL + u
            xv = xbuf[pl.ds(s * L, L)]
            _, kb = _keys(xv)
            bucket = lax.shift_right_logical(kb, 21)
            plsc.addupdate_scatter(hist, [bucket], ones)
            obuf[pl.ds(s * L, L)] = zf
        return carry

    lax.fori_loop(0, NSLICES // UNROLL, it, np.int32(0))


def _scan_hist(hist, need, nslices):
    iota = lax.iota(jnp.int32, L)

    def it(i, carry):
        found, b, c_above, acc = carry
        j = np.int32(nslices - 1) - i
        h = hist[pl.ds(j * L, L)]
        s = jnp.sum(h)
        incl = plsc.cumsum(h)
        suffix = acc + (s - incl) + h
        hit = jnp.logical_and(found == 0, (acc + s) >= need)
        mv = suffix >= need
        b_in = jnp.sum(jnp.where(mv, 1, 0)) - 1
        strict = suffix - h
        c_new = jnp.sum(jnp.where(iota == b_in, strict, 0))
        b = jnp.where(hit, j * L + b_in, b)
        c_above = jnp.where(hit, c_new, c_above)
        found = jnp.where(hit, np.int32(1), found)
        return found, b, c_above, acc + s

    z = np.int32(0)
    _, b, c_above, _ = lax.fori_loop(0, nslices, it, (z, z, z, z))
    return b, c_above


def _capture_pass(xbuf, cand, lo1_v):
    iota = lax.iota(jnp.int32, L)

    def it(i, offm1_v):
        for u in range(UNROLL):
            s = i * UNROLL + u
            xv = xbuf[pl.ds(s * L, L)]
            ki, _ = _keys(xv)
            pm = ki >= lo1_v
            incl = plsc.cumsum(jnp.where(pm, np.int32(1), np.int32(0)))
            plsc.store_scatter(cand, [offm1_v + incl], s * L + iota, mask=pm)
            offm1_v = offm1_v + plsc.all_reduce_population_count(pm)
        return offm1_v

    offm1_v = lax.fori_loop(
        0, NSLICES // UNROLL, it, jnp.full((L,), np.int32(-1), jnp.int32)
    )
    return offm1_v + 1


def _cand_count(xbuf, cand, trip, nc_v, thr_v, strict):
    iota = lax.iota(jnp.int32, L)

    def it(t, cnt_v):
        valid = (t * L + iota) < nc_v
        posv = cand[pl.ds(t * L, L)]
        xg = plsc.load_gather(xbuf, [jnp.where(valid, posv, np.int32(0))])
        ki, _ = _keys(xg)
        cmp = (ki > thr_v) if strict else (ki >= thr_v)
        m = jnp.logical_and(valid, cmp)
        return cnt_v + plsc.all_reduce_population_count(m)

    return lax.fori_loop(0, trip, it, jnp.zeros((L,), jnp.int32))


def _cand_binsearch(xbuf, cand, trip, nc_v, b1, need):
    lo0 = (b1 << 21) ^ MIN32
    lo_v = jnp.full((L,), np.int32(0), jnp.int32) + lo0
    span_v = jnp.full((L,), np.int32((1 << 21) - 1), jnp.int32)
    hi_v = lo_v + span_v
    need_v = jnp.full((L,), need, jnp.int32)

    def step(i, carry):
        lo_v, hi_v = carry
        mid_v = lo_v + ((hi_v - lo_v + 1) >> 1)
        cnt_v = _cand_count(xbuf, cand, trip, nc_v, mid_v, False)
        ok = cnt_v >= need_v
        return (jnp.where(ok, mid_v, lo_v), jnp.where(ok, hi_v, mid_v - 1))

    lo_v, _ = lax.fori_loop(0, 21, step, (lo_v, hi_v))
    return lo_v


def _fixup_pass(xbuf, obuf, cand, trip, nc_v, tsig_v, qv):
    iota = lax.iota(jnp.int32, L)

    def it(t, rv):
        valid = (t * L + iota) < nc_v
        posv = cand[pl.ds(t * L, L)]
        xg = plsc.load_gather(xbuf, [jnp.where(valid, posv, np.int32(0))])
        ki, _ = _keys(xg)
        gt = jnp.logical_and(valid, ki > tsig_v)
        eq = jnp.logical_and(valid, ki == tsig_v)
        incl = plsc.cumsum(jnp.where(eq, np.int32(1), np.int32(0)))
        take = jnp.logical_or(gt, jnp.logical_and(eq, (rv + incl) <= qv))
        val = jnp.maximum(xg, np.float32(0.0))
        plsc.store_scatter(obuf, [posv], val, mask=take)
        return rv + plsc.all_reduce_population_count(eq)

    lax.fori_loop(0, trip, it, jnp.zeros((L,), jnp.int32))


@functools.partial(
    pl.kernel,
    out_type=jax.ShapeDtypeStruct((B, N), jnp.float32),
    mesh=_MESH,
    compiler_params=pltpu.CompilerParams(needs_layout_passes=False),
    scratch_types=[
        pltpu.VMEM((N,), jnp.float32),
        pltpu.VMEM((N,), jnp.float32),
        pltpu.VMEM((N,), jnp.int32),
        pltpu.VMEM((HBUCKETS,), jnp.int32),
    ],
)
def _topk_sc(x_hbm, o_hbm, xbuf, obuf, cand, hist):
    wid = lax.axis_index("s") * 2 + lax.axis_index("c")
    for r in range(ROWS_PER_W):
        row = wid * ROWS_PER_W + r
        pltpu.sync_copy(x_hbm.at[row], xbuf)

        _zero_hist(hist, HBUCKETS // L)
        _hist_pass_a(xbuf, obuf, hist)
        b1, _ = _scan_hist(hist, np.int32(K), HBUCKETS // L)

        lo1_v = jnp.zeros((L,), jnp.int32) + ((b1 << 21) ^ MIN32)
        nc_v = _capture_pass(xbuf, cand, lo1_v)
        nc = jnp.max(nc_v)
        trip = (nc + np.int32(L - 1)) >> 4

        tsig_v = _cand_binsearch(xbuf, cand, trip, nc_v, b1, np.int32(K))
        cgt_v = _cand_count(xbuf, cand, trip, nc_v, tsig_v, True)
        qv = jnp.full((L,), np.int32(K), jnp.int32) - cgt_v

        _fixup_pass(xbuf, obuf, cand, trip, nc_v, tsig_v, qv)
        pltpu.sync_copy(obuf, o_hbm.at[row])


def kernel(x):
    return _topk_sc(x)
